# baseline (speedup 1.0000x reference)
import functools

import jax
import jax.numpy as jnp
from jax import lax
from jax.experimental import pallas as pl
from jax.experimental.pallas import tpu as pltpu
from jax.experimental.pallas import tpu_sc as plsc

_PZBL = 0.23
_A0 = 0.4685
_C = (0.02817, 0.28022, 0.50986, 0.18175)
_D = (-0.20162, -0.4029, -0.94229, -3.1998)

_SPAD = 128


def _species_prep(z_ref, qq_ref, zq_ref, zp_ref):
    z = z_ref[...]
    zq_ref[...] = z * jnp.sqrt(qq_ref[...])
    zp_ref[...] = jnp.exp(jnp.log(z) * jnp.float32(_PZBL)) * jnp.float32(1.0 / _A0)


def _make_sc_kernel(n_nodes, epw):
    info = plsc.get_sparse_core_info()
    nc, ns, L = info.num_cores, info.num_subcores, info.num_lanes

    mesh = plsc.VectorSubcoreMesh(core_axis_name="c", subcore_axis_name="s")

    @functools.partial(
        pl.kernel,
        mesh=mesh,
        compiler_params=pltpu.CompilerParams(needs_layout_passes=False),
        out_type=jax.ShapeDtypeStruct((nc * ns * epw,), jnp.float32),
        scratch_types=[
            pltpu.VMEM((n_nodes,), jnp.int32),
            pltpu.VMEM((_SPAD,), jnp.float32),
            pltpu.VMEM((_SPAD,), jnp.float32),
            pltpu.VMEM((epw,), jnp.int32),
            pltpu.VMEM((epw,), jnp.int32),
            pltpu.VMEM((epw,), jnp.float32),
            pltpu.VMEM((epw,), jnp.float32),
        ],
    )
    def zbl_sc(types_hbm, zq_hbm, zp_hbm, ei_hbm, ej_hbm, r_hbm, out_hbm,
               types_v, zq_v, zp_v, ei_v, ej_v, r_v, out_v):
        wid = lax.axis_index("s") * nc + lax.axis_index("c")
        base = wid * epw
        pltpu.sync_copy(types_hbm, types_v)
        pltpu.sync_copy(zq_hbm, zq_v)
        pltpu.sync_copy(zp_hbm, zp_v)
        pltpu.sync_copy(ei_hbm.at[pl.ds(base, epw)], ei_v)
        pltpu.sync_copy(ej_hbm.at[pl.ds(base, epw)], ej_v)
        pltpu.sync_copy(r_hbm.at[pl.ds(base, epw)], r_v)

        c1, c2, c3, c4 = (jnp.float32(c) for c in _C)
        d1, d2, d3, d4 = (jnp.float32(d) for d in _D)

        def body(k, _):
            off = k * L
            iv = ei_v[pl.ds(off, L)]
            jv = ej_v[pl.ds(off, L)]
            rv = r_v[pl.ds(off, L)]
            ti = plsc.load_gather(types_v, [iv])
            tj = plsc.load_gather(types_v, [jv])
            zi = plsc.load_gather(zq_v, [ti])
            zj = plsc.load_gather(zq_v, [tj])
            pi = plsc.load_gather(zp_v, [ti])
            pj = plsc.load_gather(zp_v, [tj])
            x = (pi + pj) * rv
            psi = (c1 * jnp.exp(d1 * x) + c2 * jnp.exp(d2 * x)
                   + c3 * jnp.exp(d3 * x) + c4 * jnp.exp(d4 * x))
            out_v[pl.ds(off, L)] = (zi * zj / rv) * psi
            return 0

        lax.fori_loop(0, epw // L, body, 0, unroll=8)
        pltpu.sync_copy(out_v, out_hbm.at[pl.ds(base, epw)])

    return zbl_sc


def kernel(Z, r, atom_types, edge_index, qqr2exesquare):
    n_edges = r.shape[0]
    n_species = Z.shape[0]
    n_nodes = atom_types.shape[0]
    assert n_edges % (32 * 16) == 0

    types32 = atom_types.astype(jnp.int32)
    ei = edge_index[0].astype(jnp.int32)
    ej = edge_index[1].astype(jnp.int32)

    z_pad = jnp.pad(Z.astype(jnp.float32), (0, _SPAD - n_species),
                    constant_values=1.0).reshape(1, _SPAD)
    qq_b = jnp.broadcast_to(jnp.float32(qqr2exesquare), (1, _SPAD))

    zq, zp = pl.pallas_call(
        _species_prep,
        out_shape=[
            jax.ShapeDtypeStruct((1, _SPAD), jnp.float32),
            jax.ShapeDtypeStruct((1, _SPAD), jnp.float32),
        ],
    )(z_pad, qq_b)
    zq = zq.reshape(_SPAD)
    zp = zp.reshape(_SPAD)

    epw = n_edges // 32
    eng = _make_sc_kernel(n_nodes, epw)(types32, zq, zp, ei, ej, r)
    return eng


# speedup vs baseline: 1.6814x; 1.6814x over previous
import functools

import jax
import jax.numpy as jnp
from jax import lax
from jax.experimental import pallas as pl
from jax.experimental.pallas import tpu as pltpu
from jax.experimental.pallas import tpu_sc as plsc

_PZBL = 0.23
_A0 = 0.4685
_C = (0.02817, 0.28022, 0.50986, 0.18175)
_D = (-0.20162, -0.4029, -0.94229, -3.1998)

_SPAD = 128


def _species_prep(z_ref, qq_ref, zq_ref, zp_ref):
    z = z_ref[...]
    zq_ref[...] = z * jnp.sqrt(qq_ref[...])
    zp_ref[...] = jnp.exp(jnp.log(z) * jnp.float32(_PZBL)) * jnp.float32(1.0 / _A0)


def _make_sc_kernel(n_nodes, epw):
    info = plsc.get_sparse_core_info()
    nc, ns, L = info.num_cores, info.num_subcores, info.num_lanes

    mesh = plsc.VectorSubcoreMesh(core_axis_name="c", subcore_axis_name="s")

    @functools.partial(
        pl.kernel,
        mesh=mesh,
        compiler_params=pltpu.CompilerParams(needs_layout_passes=False),
        out_type=jax.ShapeDtypeStruct((nc * ns * epw,), jnp.float32),
        scratch_types=[
            pltpu.VMEM((n_nodes,), jnp.int32),
            pltpu.VMEM((_SPAD,), jnp.float32),
            pltpu.VMEM((_SPAD,), jnp.float32),
            pltpu.VMEM((epw,), jnp.int32),
            pltpu.VMEM((epw,), jnp.int32),
            pltpu.VMEM((epw,), jnp.float32),
            pltpu.VMEM((epw,), jnp.float32),
        ],
    )
    def zbl_sc(types_hbm, zq_hbm, zp_hbm, ei_hbm, ej_hbm, r_hbm, out_hbm,
               types_v, zq_v, zp_v, ei_v, ej_v, r_v, out_v):
        wid = lax.axis_index("s") * nc + lax.axis_index("c")
        base = wid * epw
        pltpu.sync_copy(types_hbm, types_v)
        pltpu.sync_copy(zq_hbm, zq_v)
        pltpu.sync_copy(zp_hbm, zp_v)
        pltpu.sync_copy(ei_hbm.at[pl.ds(base, epw)], ei_v)
        pltpu.sync_copy(ej_hbm.at[pl.ds(base, epw)], ej_v)
        pltpu.sync_copy(r_hbm.at[pl.ds(base, epw)], r_v)

        c1, c2, c3, c4 = (jnp.float32(c) for c in _C)
        d1, d2, d3, d4 = (jnp.float32(d) for d in _D)

        @plsc.parallel_loop(0, epw, step=L, unroll=4)
        def body(off):
            iv = ei_v[pl.ds(off, L)]
            jv = ej_v[pl.ds(off, L)]
            rv = r_v[pl.ds(off, L)]
            ti = plsc.load_gather(types_v, [iv])
            tj = plsc.load_gather(types_v, [jv])
            zi = plsc.load_gather(zq_v, [ti])
            zj = plsc.load_gather(zq_v, [tj])
            pi = plsc.load_gather(zp_v, [ti])
            pj = plsc.load_gather(zp_v, [tj])
            x = (pi + pj) * rv
            psi = (c1 * jnp.exp(d1 * x) + c2 * jnp.exp(d2 * x)
                   + c3 * jnp.exp(d3 * x) + c4 * jnp.exp(d4 * x))
            out_v[pl.ds(off, L)] = (zi * zj / rv) * psi
        pltpu.sync_copy(out_v, out_hbm.at[pl.ds(base, epw)])

    return zbl_sc


def kernel(Z, r, atom_types, edge_index, qqr2exesquare):
    n_edges = r.shape[0]
    n_species = Z.shape[0]
    n_nodes = atom_types.shape[0]
    assert n_edges % (32 * 16) == 0

    types32 = atom_types.astype(jnp.int32)
    ei = edge_index[0].astype(jnp.int32)
    ej = edge_index[1].astype(jnp.int32)

    z_pad = jnp.pad(Z.astype(jnp.float32), (0, _SPAD - n_species),
                    constant_values=1.0).reshape(1, _SPAD)
    qq_b = jnp.broadcast_to(jnp.float32(qqr2exesquare), (1, _SPAD))

    zq, zp = pl.pallas_call(
        _species_prep,
        out_shape=[
            jax.ShapeDtypeStruct((1, _SPAD), jnp.float32),
            jax.ShapeDtypeStruct((1, _SPAD), jnp.float32),
        ],
    )(z_pad, qq_b)
    zq = zq.reshape(_SPAD)
    zp = zp.reshape(_SPAD)

    epw = n_edges // 32
    eng = _make_sc_kernel(n_nodes, epw)(types32, zq, zp, ei, ej, r)
    return eng
